# Initial kernel scaffold; baseline (speedup 1.0000x reference)
import functools

import jax
import jax.numpy as jnp
import numpy as np
from jax import lax
from jax.experimental import pallas as pl
from jax.experimental.pallas import tpu as pltpu
from jax.experimental.pallas import tpu_sc as plsc

B = 1024; NH = 50; NN = 5; D = 128; AD = 64
CAT = 18; CATP = 19; TL = 20; V = 30000; WD = 64
NODES = NH + CAT
SCALAR = float(np.sqrt(AD))

NC, NS = 2, 16
NW = NC * NS
PAIRS = B * NH
PPW = PAIRS // NW
CP = 32
ROWS = CP * TL
NDMA = 8
RPD = ROWS // NDMA
NCHUNK = PPW // CP
VPAD = V


def _sc_pool_body(table, text, mask, out, text_v, mask_v, idx_v, rows_v, out_v, sem):
    wid = lax.axis_index("s") * NC + lax.axis_index("c")

    def chunk(c, carry):
        pair0 = wid * PPW + c * CP
        base = pair0 * TL
        pltpu.sync_copy(text.at[pl.ds(base, ROWS)], text_v)
        pltpu.sync_copy(mask.at[pl.ds(base, ROWS)], mask_v)
        for j in range(NDMA):
            for l in range(RPD // 16):
                o = j * RPD + l * 16
                t = text_v[pl.ds(o, 16)]
                m = mask_v[pl.ds(o, 16)]
                idx_v[j, pl.ds(l * 16, 16)] = jnp.where(m > 0.0, t, VPAD)
        cps = [
            pltpu.async_copy(table.at[idx_v.at[j]],
                             rows_v.at[pl.ds(j * RPD, RPD)], sem)
            for j in range(NDMA)
        ]
        for c_ in cps:
            c_.wait()

        def pair(p, carry2):
            ro = p * TL
            for l in range(WD // 16):
                acc = rows_v[ro, pl.ds(l * 16, 16)]
                for t in range(1, TL):
                    acc = acc + rows_v[ro + t, pl.ds(l * 16, 16)]
                out_v[pl.ds(p * WD + l * 16, 16)] = acc
            return carry2

        lax.fori_loop(0, CP, pair, 0)
        pltpu.sync_copy(out_v, out.at[pl.ds(pair0 * WD, CP * WD)])
        return carry

    lax.fori_loop(0, NCHUNK, chunk, 0)


_sc_pool = pl.kernel(
    _sc_pool_body,
    out_type=jax.ShapeDtypeStruct((PAIRS * WD,), jnp.float32),
    mesh=plsc.VectorSubcoreMesh(core_axis_name="c", subcore_axis_name="s"),
    scratch_types=[
        pltpu.VMEM((ROWS,), jnp.int32),
        pltpu.VMEM((ROWS,), jnp.float32),
        pltpu.VMEM((NDMA, RPD), jnp.int32),
        pltpu.VMEM((ROWS, WD), jnp.float32),
        pltpu.VMEM((CP * WD,), jnp.float32),
        pltpu.SemaphoreType.DMA,
    ],
)

BB = 8


def _bmm(x, y):
    return lax.dot_general(x, y, (((2,), (1,)), ((0,), (0,))),
                           preferred_element_type=jnp.float32)


def _mm3(x, w):
    return lax.dot_general(x, w, (((2,), (0,)), ((), ())),
                           preferred_element_type=jnp.float32)


def _tc_body(psum, mask, graph, catidx, catmask, cand,
             W_news, bn, proxy, W0, b0, W1, b1, Kw, Qw, Qb,
             affW, affb, iKw, iQw, iQb, out):
    ps = psum[...]
    den = jnp.sum(mask[...], axis=2, keepdims=True)
    pooled = ps / jnp.maximum(den, 1e-6)
    hist = _mm3(pooled, W_news[...]) + bn[...][None]
    prox = jnp.broadcast_to(proxy[...][None], (BB, CAT, D))
    he = jnp.concatenate([hist, prox], axis=1)
    A = graph[...]
    h1 = jax.nn.relu(_mm3(_bmm(A, he), W0[...]) + b0[...][None]) + he
    h2 = _mm3(_bmm(A, h1), W1[...]) + b1[...][None] + h1
    gf = (h2 + he)[:, :NH, :]
    K = _mm3(gf, Kw[...])
    Qfull = _mm3(cand[...], Qw[...]) + Qb[...][None]
    ci = catidx[...]
    iotaC = lax.broadcasted_iota(jnp.int32, (BB, CATP, NH), 1)
    onehotT = (ci[:, None, :] == iotaC).astype(jnp.float32)
    cm = catmask[...]
    cm = jnp.where(lax.broadcasted_iota(jnp.int32, (BB, CATP), 1) == CATP - 1,
                   1.0, cm)
    cd = cand[...]
    outs = []
    for n in range(NN):
        q = Qfull[:, n, :]
        a = jnp.sum(K * q[:, None, :], axis=2) / SCALAR
        am = jnp.where(onehotT > 0, a[:, None, :], -1e9)
        segmax = jnp.max(am, axis=2)
        maxg = _bmm(segmax[:, None, :], onehotT)[:, 0, :]
        ea = jnp.exp(a - maxg)
        segsum = jnp.sum(jnp.where(onehotT > 0, ea[:, None, :], 0.0), axis=2)
        denom = _bmm(segsum[:, None, :], onehotT)[:, 0, :]
        alpha = ea / denom
        wn = alpha[:, :, None] * gf
        intra = _bmm(onehotT, wn)
        intra = jax.nn.relu(_mm3(intra, affW[...]) + affb[...][None]) + intra
        Kf = _mm3(intra, iKw[...])
        qf = lax.dot_general(cd[:, n, :], iQw[...], (((1,), (0,)), ((), ())),
                             preferred_element_type=jnp.float32) + iQb[...]
        s = jnp.sum(Kf * qf[:, None, :], axis=2) / SCALAR
        s = jnp.where(cm == 0, -1e9, s)
        es = jnp.exp(s - jnp.max(s, axis=1, keepdims=True))
        al = es / jnp.sum(es, axis=1, keepdims=True)
        outs.append(_bmm(al[:, None, :], intra))
    out[...] = jnp.concatenate(outs, axis=1)


def _tc_specs(batch):
    bspec = pl.BlockSpec
    full2 = lambda s: bspec(s, lambda i: (0, 0))
    in_specs = [
        bspec((BB, NH, WD), lambda i: (i, 0, 0)),
        bspec((BB, NH, TL), lambda i: (i, 0, 0)),
        bspec((BB, NODES, NODES), lambda i: (i, 0, 0)),
        bspec((BB, NH), lambda i: (i, 0)),
        bspec((BB, CATP), lambda i: (i, 0)),
        bspec((BB, NN, D), lambda i: (i, 0, 0)),
        full2((WD, D)), full2((1, D)), full2((CAT, D)),
        full2((D, D)), full2((1, D)), full2((D, D)), full2((1, D)),
        full2((D, AD)), full2((D, AD)), full2((1, AD)),
        full2((D, D)), full2((1, D)),
        full2((D, AD)), full2((D, AD)), full2((1, AD)),
    ]
    out_specs = bspec((BB, NN, D), lambda i: (i, 0, 0))
    return dict(
        grid=(batch // BB,),
        in_specs=in_specs,
        out_specs=out_specs,
        out_shape=jax.ShapeDtypeStruct((batch, NN, D), jnp.float32),
        compiler_params=pltpu.CompilerParams(
            dimension_semantics=("arbitrary",)),
    )


def kernel(user_title_text, user_title_mask, user_title_entity,
           user_content_text, user_content_mask, user_content_entity,
           user_category, user_subCategory, user_history_mask,
           user_history_graph, user_history_category_mask,
           user_history_category_indices, user_embedding,
           candidate_news_representation, word_emb, W_news, b_news,
           proxy_emb, gcn_W0, gcn_b0, gcn_W1, gcn_b1, Kw, Qw, Qb,
           aff_W, aff_b, inter_Kw, inter_Qw, inter_Qb):
    batch = user_title_text.shape[0]
    table = jnp.concatenate(
        [word_emb, jnp.zeros((8, WD), jnp.float32)], axis=0)
    text_flat = user_title_text.reshape(-1).astype(jnp.int32)
    mask_flat = user_title_mask.reshape(-1)
    psum = _sc_pool(table, text_flat, mask_flat).reshape(batch, NH, WD)

    tc = pl.pallas_call(_tc_body, **_tc_specs(batch))
    out = tc(psum, user_title_mask, user_history_graph,
             user_history_category_indices.astype(jnp.int32),
             user_history_category_mask, candidate_news_representation,
             W_news, b_news.reshape(1, D), proxy_emb,
             gcn_W0, gcn_b0.reshape(1, D), gcn_W1, gcn_b1.reshape(1, D),
             Kw, Qw, Qb.reshape(1, AD), aff_W, aff_b.reshape(1, D),
             inter_Kw, inter_Qw, inter_Qb.reshape(1, AD))
    return out


# speedup vs baseline: 1.6429x; 1.6429x over previous
import functools

import jax
import jax.numpy as jnp
import numpy as np
from jax import lax
from jax.experimental import pallas as pl
from jax.experimental.pallas import tpu as pltpu
from jax.experimental.pallas import tpu_sc as plsc

B = 1024; NH = 50; NN = 5; D = 128; AD = 64
CAT = 18; CATP = 19; TL = 20; V = 30000; WD = 64
NODES = NH + CAT
SCALAR = float(np.sqrt(AD))

NC, NS = 2, 16
NW = NC * NS
PAIRS = B * NH
PPW = PAIRS // NW
CP = 32
ROWS = CP * TL
NDMA = 8
RPD = ROWS // NDMA
NCHUNK = PPW // CP
VPAD = V


def _sc_pool_body(table, text, mask, out, text_v, mask_v, idx_v, rows_v, out_v, sem):
    wid = lax.axis_index("s") * NC + lax.axis_index("c")

    def chunk(c, carry):
        pair0 = wid * PPW + c * CP
        base = pair0 * TL
        pltpu.sync_copy(text.at[pl.ds(base, ROWS)], text_v)
        pltpu.sync_copy(mask.at[pl.ds(base, ROWS)], mask_v)
        for j in range(NDMA):
            for l in range(RPD // 16):
                o = j * RPD + l * 16
                t = text_v[pl.ds(o, 16)]
                m = mask_v[pl.ds(o, 16)]
                idx_v[j, pl.ds(l * 16, 16)] = jnp.where(m > 0.0, t, VPAD)
        cps = [
            pltpu.async_copy(table.at[idx_v.at[j]],
                             rows_v.at[pl.ds(j * RPD, RPD)], sem)
            for j in range(NDMA)
        ]
        for c_ in cps:
            c_.wait()

        def pair(p, carry2):
            ro = p * TL
            for l in range(WD // 16):
                acc = rows_v[ro, pl.ds(l * 16, 16)]
                for t in range(1, TL):
                    acc = acc + rows_v[ro + t, pl.ds(l * 16, 16)]
                out_v[pl.ds(p * WD + l * 16, 16)] = acc
            return carry2

        lax.fori_loop(0, CP, pair, 0)
        pltpu.sync_copy(out_v, out.at[pl.ds(pair0 * WD, CP * WD)])
        return carry

    lax.fori_loop(0, NCHUNK, chunk, 0)


@functools.cache
def _sc_pool():
    return pl.kernel(
        _sc_pool_body,
        out_type=jax.ShapeDtypeStruct((PAIRS * WD,), jnp.float32),
        mesh=plsc.VectorSubcoreMesh(core_axis_name="c", subcore_axis_name="s",
                                    num_cores=NC, num_subcores=NS),
        scratch_types=[
            pltpu.VMEM((ROWS,), jnp.int32),
            pltpu.VMEM((ROWS,), jnp.float32),
            pltpu.VMEM((NDMA, RPD), jnp.int32),
            pltpu.VMEM((ROWS, WD), jnp.float32),
            pltpu.VMEM((CP * WD,), jnp.float32),
            pltpu.SemaphoreType.DMA,
        ],
        compiler_params=pltpu.CompilerParams(use_tc_tiling_on_sc=False),
    )

BB = 8


def _bmm(x, y):
    return lax.dot_general(x, y, (((2,), (1,)), ((0,), (0,))),
                           preferred_element_type=jnp.float32)


def _mm3(x, w):
    return lax.dot_general(x, w, (((2,), (0,)), ((), ())),
                           preferred_element_type=jnp.float32)


def _tc_body(psum, mask, graph, catidx, catmask, cand,
             W_news, bn, proxy, W0, b0, W1, b1, Kw, Qw, Qb,
             affW, affb, iKw, iQw, iQb, out):
    ps = psum[...]
    den = jnp.sum(mask[...], axis=2, keepdims=True)
    pooled = ps / jnp.maximum(den, 1e-6)
    hist = _mm3(pooled, W_news[...]) + bn[...][None]
    prox = jnp.broadcast_to(proxy[...][None], (BB, CAT, D))
    he = jnp.concatenate([hist, prox], axis=1)
    A = graph[...]
    h1 = jax.nn.relu(_mm3(_bmm(A, he), W0[...]) + b0[...][None]) + he
    h2 = _mm3(_bmm(A, h1), W1[...]) + b1[...][None] + h1
    gf = (h2 + he)[:, :NH, :]
    K = _mm3(gf, Kw[...])
    Qfull = _mm3(cand[...], Qw[...]) + Qb[...][None]
    ci = catidx[...]
    iotaC = lax.broadcasted_iota(jnp.int32, (BB, CATP, NH), 1)
    onehotT = (ci[:, None, :] == iotaC).astype(jnp.float32)
    cm = catmask[...]
    cm = jnp.where(lax.broadcasted_iota(jnp.int32, (BB, CATP), 1) == CATP - 1,
                   1.0, cm)
    cd = cand[...]
    outs = []
    for n in range(NN):
        q = Qfull[:, n, :]
        a = jnp.sum(K * q[:, None, :], axis=2) / SCALAR
        am = jnp.where(onehotT > 0, a[:, None, :], -1e9)
        segmax = jnp.max(am, axis=2)
        maxg = _bmm(segmax[:, None, :], onehotT)[:, 0, :]
        ea = jnp.exp(a - maxg)
        segsum = jnp.sum(jnp.where(onehotT > 0, ea[:, None, :], 0.0), axis=2)
        denom = _bmm(segsum[:, None, :], onehotT)[:, 0, :]
        alpha = ea / denom
        wn = alpha[:, :, None] * gf
        intra = _bmm(onehotT, wn)
        intra = jax.nn.relu(_mm3(intra, affW[...]) + affb[...][None]) + intra
        Kf = _mm3(intra, iKw[...])
        qf = lax.dot_general(cd[:, n, :], iQw[...], (((1,), (0,)), ((), ())),
                             preferred_element_type=jnp.float32) + iQb[...]
        s = jnp.sum(Kf * qf[:, None, :], axis=2) / SCALAR
        s = jnp.where(cm == 0, -1e9, s)
        es = jnp.exp(s - jnp.max(s, axis=1, keepdims=True))
        al = es / jnp.sum(es, axis=1, keepdims=True)
        outs.append(_bmm(al[:, None, :], intra))
    out[...] = jnp.concatenate(outs, axis=1)


def _tc_specs(batch):
    bspec = pl.BlockSpec
    full2 = lambda s: bspec(s, lambda i: (0, 0))
    in_specs = [
        bspec((BB, NH, WD), lambda i: (i, 0, 0)),
        bspec((BB, NH, TL), lambda i: (i, 0, 0)),
        bspec((BB, NODES, NODES), lambda i: (i, 0, 0)),
        bspec((BB, NH), lambda i: (i, 0)),
        bspec((BB, CATP), lambda i: (i, 0)),
        bspec((BB, NN, D), lambda i: (i, 0, 0)),
        full2((WD, D)), full2((1, D)), full2((CAT, D)),
        full2((D, D)), full2((1, D)), full2((D, D)), full2((1, D)),
        full2((D, AD)), full2((D, AD)), full2((1, AD)),
        full2((D, D)), full2((1, D)),
        full2((D, AD)), full2((D, AD)), full2((1, AD)),
    ]
    out_specs = bspec((BB, NN, D), lambda i: (i, 0, 0))
    return dict(
        grid=(batch // BB,),
        in_specs=in_specs,
        out_specs=out_specs,
        out_shape=jax.ShapeDtypeStruct((batch, NN, D), jnp.float32),
        compiler_params=pltpu.CompilerParams(
            dimension_semantics=("arbitrary",)),
    )


def kernel(user_title_text, user_title_mask, user_title_entity,
           user_content_text, user_content_mask, user_content_entity,
           user_category, user_subCategory, user_history_mask,
           user_history_graph, user_history_category_mask,
           user_history_category_indices, user_embedding,
           candidate_news_representation, word_emb, W_news, b_news,
           proxy_emb, gcn_W0, gcn_b0, gcn_W1, gcn_b1, Kw, Qw, Qb,
           aff_W, aff_b, inter_Kw, inter_Qw, inter_Qb):
    batch = user_title_text.shape[0]
    table = jnp.concatenate(
        [word_emb, jnp.zeros((8, WD), jnp.float32)], axis=0)
    text_flat = user_title_text.reshape(-1).astype(jnp.int32)
    mask_flat = user_title_mask.reshape(-1)
    psum = _sc_pool()(table, text_flat, mask_flat).reshape(batch, NH, WD)

    tc = pl.pallas_call(_tc_body, **_tc_specs(batch))
    out = tc(psum, user_title_mask, user_history_graph,
             user_history_category_indices.astype(jnp.int32),
             user_history_category_mask, candidate_news_representation,
             W_news, b_news.reshape(1, D), proxy_emb,
             gcn_W0, gcn_b0.reshape(1, D), gcn_W1, gcn_b1.reshape(1, D),
             Kw, Qw, Qb.reshape(1, AD), aff_W, aff_b.reshape(1, D),
             inter_Kw, inter_Qw, inter_Qb.reshape(1, AD))
    return out
